# Initial kernel scaffold; baseline (speedup 1.0000x reference)
import jax
import jax.numpy as jnp
from jax.experimental import pallas as pl


def kernel(x, gate_w, fc1_w, fc2_w, fc3_w):
    raise NotImplementedError("write your pallas kernel here")


# speedup vs baseline: 1.3241x; 1.3241x over previous
import functools

import jax
import jax.numpy as jnp
from jax import lax
from jax.experimental import pallas as pl
from jax.experimental.pallas import tpu as pltpu
from jax.experimental.pallas import tpu_sc as plsc

TOP_K = 2
BLK = 256
TB = 512
NUM_CORES = 2
NUM_SUBCORES = 16
NW = NUM_CORES * NUM_SUBCORES


def _router_body(x_ref, gw_ref, i0_ref, i1_ref, p0_ref, p1_ref):
    xb = x_ref[...]
    gw = gw_ref[...]
    s = lax.dot_general(xb, gw, (((1,), (1,)), ((), ())),
                        preferred_element_type=jnp.float32)
    e = s.shape[1]
    cols = lax.broadcasted_iota(jnp.int32, s.shape, 1)
    m0 = jnp.max(s, axis=1)
    i0 = jnp.min(jnp.where(s == m0[:, None], cols, e), axis=1)
    s2 = jnp.where(cols == i0[:, None], -jnp.inf, s)
    m1 = jnp.max(s2, axis=1)
    i1 = jnp.min(jnp.where(s2 == m1[:, None], cols, e), axis=1)
    p0 = 1.0 / (1.0 + jnp.exp(m1 - m0))
    i0_ref[...] = i0[:, None].astype(jnp.int32)
    i1_ref[...] = i1[:, None].astype(jnp.int32)
    p0_ref[...] = p0[:, None]
    p1_ref[...] = (1.0 - p0)[:, None]


def _router(x2d, gate_w):
    t, d = x2d.shape
    e = gate_w.shape[0]
    outs = pl.pallas_call(
        _router_body,
        grid=(t // TB,),
        in_specs=[
            pl.BlockSpec((TB, d), lambda b: (b, 0)),
            pl.BlockSpec((e, d), lambda b: (0, 0)),
        ],
        out_specs=[pl.BlockSpec((TB, 1), lambda b: (b, 0))] * 4,
        out_shape=[
            jax.ShapeDtypeStruct((t, 1), jnp.int32),
            jax.ShapeDtypeStruct((t, 1), jnp.int32),
            jax.ShapeDtypeStruct((t, 1), jnp.float32),
            jax.ShapeDtypeStruct((t, 1), jnp.float32),
        ],
    )(x2d, gate_w)
    return [o[:, 0] for o in outs]


def _sc_gather(x2d, row_token, p):
    t, d = x2d.shape
    rpw = p // NW
    ch = 32
    mesh = plsc.VectorSubcoreMesh(core_axis_name="c", subcore_axis_name="s")

    @functools.partial(
        pl.kernel,
        out_type=jax.ShapeDtypeStruct((p, d), jnp.float32),
        mesh=mesh,
        scratch_types=[
            pltpu.VMEM((ch,), jnp.int32),
            pltpu.VMEM((ch, d), jnp.float32),
            pltpu.SemaphoreType.DMA,
        ],
    )
    def gather_k(x_hbm, tok_hbm, out_hbm, idx_v, rows_v, sem):
        wid = lax.axis_index("s") * NUM_CORES + lax.axis_index("c")
        base = wid * rpw

        def chunk(i, carry):
            off = base + i * ch
            pltpu.sync_copy(tok_hbm.at[pl.ds(off, ch)], idx_v)
            pltpu.async_copy(x_hbm.at[idx_v], rows_v, sem).wait()
            pltpu.sync_copy(rows_v, out_hbm.at[pl.ds(off, ch)])
            return carry

        lax.fori_loop(0, rpw // ch, chunk, 0)

    return gather_k(x2d, row_token)


def _sc_combine(ys, pos0, pos1):
    p, d = ys.shape
    t = pos0.shape[0]
    tpw = t // NW
    ch = 16
    mesh = plsc.VectorSubcoreMesh(core_axis_name="c", subcore_axis_name="s")

    @functools.partial(
        pl.kernel,
        out_type=jax.ShapeDtypeStruct((t, d), jnp.float32),
        mesh=mesh,
        scratch_types=[
            pltpu.VMEM((ch,), jnp.int32),
            pltpu.VMEM((ch,), jnp.int32),
            pltpu.VMEM((ch, d), jnp.float32),
            pltpu.VMEM((ch, d), jnp.float32),
            pltpu.SemaphoreType.DMA,
            pltpu.SemaphoreType.DMA,
        ],
    )
    def combine_k(y_hbm, p0_hbm, p1_hbm, out_hbm, i0_v, i1_v, a_v, b_v,
                  sem0, sem1):
        wid = lax.axis_index("s") * NUM_CORES + lax.axis_index("c")
        base = wid * tpw

        def chunk(i, carry):
            off = base + i * ch
            pltpu.sync_copy(p0_hbm.at[pl.ds(off, ch)], i0_v)
            pltpu.sync_copy(p1_hbm.at[pl.ds(off, ch)], i1_v)
            cp0 = pltpu.async_copy(y_hbm.at[i0_v], a_v, sem0)
            cp1 = pltpu.async_copy(y_hbm.at[i1_v], b_v, sem1)
            cp0.wait()
            cp1.wait()

            def add_col(c, carry2):
                for r in range(ch):
                    sl = pl.ds(c * 16, 16)
                    a_v[r, sl] = a_v[r, sl] + b_v[r, sl]
                return carry2

            lax.fori_loop(0, d // 16, add_col, 0)
            pltpu.sync_copy(a_v, out_hbm.at[pl.ds(off, ch)])
            return carry

        lax.fori_loop(0, tpw // ch, chunk, 0)

    return combine_k(ys, pos0, pos1)


def _mlp_body(be_ref, xs_ref, w1_ref, w2_ref, w3_ref, ps_ref, out_ref):
    xb = xs_ref[...].astype(jnp.bfloat16)
    dn = (((1,), (1,)), ((), ()))
    h1 = lax.dot_general(xb, w1_ref[0], dn, preferred_element_type=jnp.float32)
    h2 = lax.dot_general(xb, w2_ref[0], dn, preferred_element_type=jnp.float32)
    h = (h1 * jax.nn.sigmoid(h1) * h2).astype(jnp.bfloat16)
    y = lax.dot_general(h, w3_ref[0], dn, preferred_element_type=jnp.float32)
    out_ref[...] = y * ps_ref[...]


def _grouped_mlp(block_expert, xs, fc1_w, fc2_w, fc3_w, row_scale):
    p, d = xs.shape
    e, f, _ = fc1_w.shape
    nblk = p // BLK
    grid_spec = pltpu.PrefetchScalarGridSpec(
        num_scalar_prefetch=1,
        grid=(nblk,),
        in_specs=[
            pl.BlockSpec((BLK, d), lambda b, be: (b, 0)),
            pl.BlockSpec((1, f, d), lambda b, be: (be[b], 0, 0)),
            pl.BlockSpec((1, f, d), lambda b, be: (be[b], 0, 0)),
            pl.BlockSpec((1, d, f), lambda b, be: (be[b], 0, 0)),
            pl.BlockSpec((BLK, 1), lambda b, be: (b, 0)),
        ],
        out_specs=pl.BlockSpec((BLK, d), lambda b, be: (b, 0)),
    )
    return pl.pallas_call(
        _mlp_body,
        grid_spec=grid_spec,
        out_shape=jax.ShapeDtypeStruct((p, d), jnp.float32),
    )(block_expert, xs, fc1_w, fc2_w, fc3_w, row_scale)


def kernel(x, gate_w, fc1_w, fc2_w, fc3_w):
    b, s, d = x.shape
    e = gate_w.shape[0]
    t = b * s
    p = TOP_K * t + e * BLK
    x2d = x.reshape(t, d)

    i0, i1, p0, p1 = _router(x2d, gate_w)

    e_flat = jnp.stack([i0, i1], axis=1).reshape(-1)
    p_flat = jnp.stack([p0, p1], axis=1).reshape(-1)
    oh = (e_flat[:, None] == jnp.arange(e, dtype=jnp.int32)[None, :]).astype(
        jnp.int32)
    cum = jnp.cumsum(oh, axis=0)
    rank = jnp.sum((cum - oh) * oh, axis=1)
    counts = cum[-1]
    padded = ((counts + BLK - 1) // BLK) * BLK
    starts = jnp.concatenate(
        [jnp.zeros((1,), jnp.int32), jnp.cumsum(padded)[:-1].astype(jnp.int32)])
    pos = starts[e_flat] + rank
    row_token = jnp.zeros((p,), jnp.int32).at[pos].set(
        jnp.repeat(jnp.arange(t, dtype=jnp.int32), TOP_K))
    row_scale = jnp.zeros((p, 1), jnp.float32).at[pos].set(p_flat[:, None])
    pos2 = pos.reshape(t, TOP_K).astype(jnp.int32)
    block_expert = (jnp.searchsorted(
        starts, jnp.arange(p // BLK, dtype=jnp.int32) * BLK, side="right")
        .astype(jnp.int32) - 1)

    xs = _sc_gather(x2d, row_token, p)
    ys = _grouped_mlp(block_expert, xs,
                      fc1_w.astype(jnp.bfloat16),
                      fc2_w.astype(jnp.bfloat16),
                      fc3_w.astype(jnp.bfloat16), row_scale)
    out2d = _sc_combine(ys, pos2[:, 0], pos2[:, 1])
    return out2d.reshape(b, s, d)


# speedup vs baseline: 1.5069x; 1.1381x over previous
import functools

import jax
import jax.numpy as jnp
from jax import lax
from jax.experimental import pallas as pl
from jax.experimental.pallas import tpu as pltpu
from jax.experimental.pallas import tpu_sc as plsc

TOP_K = 2
BLK = 256
TB = 512
NUM_CORES = 2
NUM_SUBCORES = 16
NW = NUM_CORES * NUM_SUBCORES


def _router_body(x_ref, gw_ref, i0_ref, i1_ref, p0_ref, p1_ref):
    xb = x_ref[...]
    gw = gw_ref[...]
    s = lax.dot_general(xb, gw, (((1,), (1,)), ((), ())),
                        preferred_element_type=jnp.float32)
    e = s.shape[1]
    cols = lax.broadcasted_iota(jnp.int32, s.shape, 1)
    m0 = jnp.max(s, axis=1)
    i0 = jnp.min(jnp.where(s == m0[:, None], cols, e), axis=1)
    s2 = jnp.where(cols == i0[:, None], -jnp.inf, s)
    m1 = jnp.max(s2, axis=1)
    i1 = jnp.min(jnp.where(s2 == m1[:, None], cols, e), axis=1)
    p0 = 1.0 / (1.0 + jnp.exp(m1 - m0))
    i0_ref[...] = i0[:, None].astype(jnp.int32)
    i1_ref[...] = i1[:, None].astype(jnp.int32)
    p0_ref[...] = p0[:, None]
    p1_ref[...] = (1.0 - p0)[:, None]


def _router(x2d, gate_w):
    t, d = x2d.shape
    e = gate_w.shape[0]
    outs = pl.pallas_call(
        _router_body,
        grid=(t // TB,),
        in_specs=[
            pl.BlockSpec((TB, d), lambda b: (b, 0)),
            pl.BlockSpec((e, d), lambda b: (0, 0)),
        ],
        out_specs=[pl.BlockSpec((TB, 1), lambda b: (b, 0))] * 4,
        out_shape=[
            jax.ShapeDtypeStruct((t, 1), jnp.int32),
            jax.ShapeDtypeStruct((t, 1), jnp.int32),
            jax.ShapeDtypeStruct((t, 1), jnp.float32),
            jax.ShapeDtypeStruct((t, 1), jnp.float32),
        ],
    )(x2d, gate_w)
    return [o[:, 0] for o in outs]


def _sc_gather(x2d, row_token, p):
    t, d = x2d.shape
    rpw = p // NW
    ch = 16
    n = rpw // ch
    mesh = plsc.VectorSubcoreMesh(core_axis_name="c", subcore_axis_name="s")

    @functools.partial(
        pl.kernel,
        out_type=jax.ShapeDtypeStruct((p, d), jnp.float32),
        mesh=mesh,
        scratch_types=[
            pltpu.VMEM((2, ch), jnp.int32),
            pltpu.VMEM((ch, d), jnp.float32),
            pltpu.VMEM((ch, d), jnp.float32),
            pltpu.SemaphoreType.DMA,
            pltpu.SemaphoreType.DMA,
            pltpu.SemaphoreType.DMA,
            pltpu.SemaphoreType.DMA,
        ],
    )
    def gather_k(x_hbm, tok_hbm, out_hbm, idx_v, rows0, rows1,
                 sg0, sg1, ss0, ss1):
        wid = lax.axis_index("s") * NUM_CORES + lax.axis_index("c")
        base = wid * rpw
        rows = (rows0, rows1)
        sg = (sg0, sg1)
        ss = (ss0, ss1)
        gathers = [None, None]
        stores = [None, None]

        def start(i):
            b = i & 1
            pltpu.sync_copy(tok_hbm.at[pl.ds(base + i * ch, ch)],
                            idx_v.at[b])
            gathers[b] = pltpu.async_copy(x_hbm.at[idx_v.at[b]], rows[b],
                                          sg[b])

        start(0)
        for i in range(n):
            b = i & 1
            if i + 1 < n:
                if stores[1 - b] is not None:
                    stores[1 - b].wait()
                start(i + 1)
            gathers[b].wait()
            stores[b] = pltpu.async_copy(
                rows[b], out_hbm.at[pl.ds(base + i * ch, ch)], ss[b])
        if n > 1:
            stores[n & 1].wait()
        stores[(n - 1) & 1].wait()

    return gather_k(x2d, row_token)


def _sc_combine(ys, pos0, pos1):
    p, d = ys.shape
    t = pos0.shape[0]
    tpw = t // NW
    ch = 8
    n = tpw // ch
    mesh = plsc.VectorSubcoreMesh(core_axis_name="c", subcore_axis_name="s")

    @functools.partial(
        pl.kernel,
        out_type=jax.ShapeDtypeStruct((t, d), jnp.float32),
        mesh=mesh,
        scratch_types=[
            pltpu.VMEM((2, ch), jnp.int32),
            pltpu.VMEM((2, ch), jnp.int32),
            pltpu.VMEM((ch, d), jnp.float32),
            pltpu.VMEM((ch, d), jnp.float32),
            pltpu.VMEM((ch, d), jnp.float32),
            pltpu.VMEM((ch, d), jnp.float32),
            pltpu.SemaphoreType.DMA,
            pltpu.SemaphoreType.DMA,
            pltpu.SemaphoreType.DMA,
            pltpu.SemaphoreType.DMA,
            pltpu.SemaphoreType.DMA,
            pltpu.SemaphoreType.DMA,
        ],
    )
    def combine_k(y_hbm, p0_hbm, p1_hbm, out_hbm, i0_v, i1_v,
                  a0, a1, b0, b1, sa0, sa1, sb0, sb1, ss0, ss1):
        wid = lax.axis_index("s") * NUM_CORES + lax.axis_index("c")
        base = wid * tpw
        av = (a0, a1)
        bv = (b0, b1)
        sa = (sa0, sa1)
        sb = (sb0, sb1)
        ss = (ss0, ss1)
        ga = [None, None]
        gb = [None, None]
        st = [None, None]

        def start(i):
            k = i & 1
            off = base + i * ch
            pltpu.sync_copy(p0_hbm.at[pl.ds(off, ch)], i0_v.at[k])
            pltpu.sync_copy(p1_hbm.at[pl.ds(off, ch)], i1_v.at[k])
            ga[k] = pltpu.async_copy(y_hbm.at[i0_v.at[k]], av[k], sa[k])
            gb[k] = pltpu.async_copy(y_hbm.at[i1_v.at[k]], bv[k], sb[k])

        start(0)
        for i in range(n):
            k = i & 1
            if i + 1 < n:
                if st[1 - k] is not None:
                    st[1 - k].wait()
                start(i + 1)
            ga[k].wait()
            gb[k].wait()
            a_ref, b_ref = av[k], bv[k]

            def add_col(c, carry, a_ref=a_ref, b_ref=b_ref):
                sl = pl.ds(c * 16, 16)
                for r in range(ch):
                    a_ref[r, sl] = a_ref[r, sl] + b_ref[r, sl]
                return carry

            lax.fori_loop(0, d // 16, add_col, 0)
            st[k] = pltpu.async_copy(
                a_ref, out_hbm.at[pl.ds(base + i * ch, ch)], ss[k])
        if n > 1:
            st[n & 1].wait()
        st[(n - 1) & 1].wait()

    return combine_k(ys, pos0, pos1)


def _mlp1_body(be_ref, xs_ref, w1_ref, w2_ref, h_ref):
    xb = xs_ref[...].astype(jnp.bfloat16)
    w1 = w1_ref[0].astype(jnp.bfloat16)
    w2 = w2_ref[0].astype(jnp.bfloat16)
    dn = (((1,), (1,)), ((), ()))
    h1 = lax.dot_general(xb, w1, dn, preferred_element_type=jnp.float32)
    h2 = lax.dot_general(xb, w2, dn, preferred_element_type=jnp.float32)
    h_ref[...] = (h1 * jax.nn.sigmoid(h1) * h2).astype(jnp.bfloat16)


def _mlp2_body(be_ref, h_ref, w3_ref, ps_ref, out_ref):
    hb = h_ref[...]
    w3 = w3_ref[0].astype(jnp.bfloat16)
    y = lax.dot_general(hb, w3, (((1,), (1,)), ((), ())),
                        preferred_element_type=jnp.float32)
    out_ref[...] = y * ps_ref[...]


def _grouped_mlp(block_expert, xs, fc1_w, fc2_w, fc3_w, row_scale):
    p, d = xs.shape
    e, f, _ = fc1_w.shape
    nblk = p // BLK
    hs = pl.pallas_call(
        _mlp1_body,
        grid_spec=pltpu.PrefetchScalarGridSpec(
            num_scalar_prefetch=1,
            grid=(nblk,),
            in_specs=[
                pl.BlockSpec((BLK, d), lambda b, be: (b, 0)),
                pl.BlockSpec((1, f, d), lambda b, be: (be[b], 0, 0)),
                pl.BlockSpec((1, f, d), lambda b, be: (be[b], 0, 0)),
            ],
            out_specs=pl.BlockSpec((BLK, f), lambda b, be: (b, 0)),
        ),
        out_shape=jax.ShapeDtypeStruct((p, f), jnp.bfloat16),
    )(block_expert, xs, fc1_w, fc2_w)
    return pl.pallas_call(
        _mlp2_body,
        grid_spec=pltpu.PrefetchScalarGridSpec(
            num_scalar_prefetch=1,
            grid=(nblk,),
            in_specs=[
                pl.BlockSpec((BLK, f), lambda b, be: (b, 0)),
                pl.BlockSpec((1, d, f), lambda b, be: (be[b], 0, 0)),
                pl.BlockSpec((BLK, 1), lambda b, be: (b, 0)),
            ],
            out_specs=pl.BlockSpec((BLK, d), lambda b, be: (b, 0)),
        ),
        out_shape=jax.ShapeDtypeStruct((p, d), jnp.float32),
    )(block_expert, hs, fc3_w, row_scale)


def kernel(x, gate_w, fc1_w, fc2_w, fc3_w):
    b, s, d = x.shape
    e = gate_w.shape[0]
    t = b * s
    p = TOP_K * t + e * BLK
    x2d = x.reshape(t, d)

    i0, i1, p0, p1 = _router(x2d, gate_w)

    e_flat = jnp.stack([i0, i1], axis=1).reshape(-1)
    p_flat = jnp.stack([p0, p1], axis=1).reshape(-1)
    oh = (e_flat[:, None] == jnp.arange(e, dtype=jnp.int32)[None, :]).astype(
        jnp.int32)
    cum = jnp.cumsum(oh, axis=0)
    rank = jnp.sum((cum - oh) * oh, axis=1)
    counts = cum[-1]
    padded = ((counts + BLK - 1) // BLK) * BLK
    starts = jnp.concatenate(
        [jnp.zeros((1,), jnp.int32), jnp.cumsum(padded)[:-1].astype(jnp.int32)])
    pos = starts[e_flat] + rank
    row_token = jnp.zeros((p,), jnp.int32).at[pos].set(
        jnp.repeat(jnp.arange(t, dtype=jnp.int32), TOP_K))
    row_scale = jnp.zeros((p, 1), jnp.float32).at[pos].set(p_flat[:, None])
    pos2 = pos.reshape(t, TOP_K).astype(jnp.int32)
    block_expert = (jnp.searchsorted(
        starts, jnp.arange(p // BLK, dtype=jnp.int32) * BLK, side="right")
        .astype(jnp.int32) - 1)

    xs = _sc_gather(x2d, row_token, p)
    ys = _grouped_mlp(block_expert, xs, fc1_w, fc2_w, fc3_w, row_scale)
    out2d = _sc_combine(ys, pos2[:, 0], pos2[:, 1])
    return out2d.reshape(b, s, d)


# speedup vs baseline: 1.5248x; 1.0118x over previous
import functools

import jax
import jax.numpy as jnp
from jax import lax
from jax.experimental import pallas as pl
from jax.experimental.pallas import tpu as pltpu
from jax.experimental.pallas import tpu_sc as plsc

TOP_K = 2
BLK = 256
TB = 512
NUM_CORES = 2
NUM_SUBCORES = 16
NW = NUM_CORES * NUM_SUBCORES


def _router_body(x_ref, gw_ref, i0_ref, i1_ref, p0_ref, p1_ref):
    xb = x_ref[...]
    gw = gw_ref[...]
    s = lax.dot_general(xb, gw, (((1,), (1,)), ((), ())),
                        preferred_element_type=jnp.float32)
    e = s.shape[1]
    cols = lax.broadcasted_iota(jnp.int32, s.shape, 1)
    m0 = jnp.max(s, axis=1)
    i0 = jnp.min(jnp.where(s == m0[:, None], cols, e), axis=1)
    s2 = jnp.where(cols == i0[:, None], -jnp.inf, s)
    m1 = jnp.max(s2, axis=1)
    i1 = jnp.min(jnp.where(s2 == m1[:, None], cols, e), axis=1)
    p0 = 1.0 / (1.0 + jnp.exp(m1 - m0))
    i0_ref[...] = i0[:, None].astype(jnp.int32)
    i1_ref[...] = i1[:, None].astype(jnp.int32)
    p0_ref[...] = p0[:, None]
    p1_ref[...] = (1.0 - p0)[:, None]


def _router(x2d, gate_w):
    t, d = x2d.shape
    e = gate_w.shape[0]
    outs = pl.pallas_call(
        _router_body,
        grid=(t // TB,),
        in_specs=[
            pl.BlockSpec((TB, d), lambda b: (b, 0)),
            pl.BlockSpec((e, d), lambda b: (0, 0)),
        ],
        out_specs=[pl.BlockSpec((TB, 1), lambda b: (b, 0))] * 4,
        out_shape=[
            jax.ShapeDtypeStruct((t, 1), jnp.int32),
            jax.ShapeDtypeStruct((t, 1), jnp.int32),
            jax.ShapeDtypeStruct((t, 1), jnp.float32),
            jax.ShapeDtypeStruct((t, 1), jnp.float32),
        ],
    )(x2d, gate_w)
    return [o[:, 0] for o in outs]


def _sc_gather(x2d, row_token, p):
    t, d = x2d.shape
    rpw = p // NW
    ch = 8
    nbuf = 6
    n = rpw // ch
    mesh = plsc.VectorSubcoreMesh(core_axis_name="c", subcore_axis_name="s")

    @functools.partial(
        pl.kernel,
        out_type=jax.ShapeDtypeStruct((p, d), jnp.float32),
        mesh=mesh,
        scratch_types=[pltpu.VMEM((rpw,), jnp.int32)]
        + [pltpu.VMEM((ch, d), jnp.float32)] * nbuf
        + [pltpu.SemaphoreType.DMA] * (2 * nbuf),
    )
    def gather_k(x_hbm, tok_hbm, out_hbm, idx_all, *bufs_sems):
        rows = bufs_sems[:nbuf]
        sg = bufs_sems[nbuf:2 * nbuf]
        ss = bufs_sems[2 * nbuf:]
        wid = lax.axis_index("s") * NUM_CORES + lax.axis_index("c")
        base = wid * rpw
        pltpu.sync_copy(tok_hbm.at[pl.ds(base, rpw)], idx_all)
        gat = [None] * nbuf
        st = [None] * nbuf

        def start(i):
            b = i % nbuf
            gat[b] = pltpu.async_copy(
                x_hbm.at[idx_all.at[pl.ds(i * ch, ch)]], rows[b], sg[b])

        for i in range(min(nbuf, n)):
            start(i)
        for i in range(n):
            b = i % nbuf
            gat[b].wait()
            st[b] = pltpu.async_copy(
                rows[b], out_hbm.at[pl.ds(base + i * ch, ch)], ss[b])
            j = i + nbuf
            if j < n:
                st[b].wait()
                start(j)
        for i in range(max(0, n - nbuf), n):
            st[i % nbuf].wait()

    return gather_k(x2d, row_token)


def _sc_combine(ys, pos0, pos1):
    p, d = ys.shape
    t = pos0.shape[0]
    tpw = t // NW
    ch = 8
    nbuf = 3
    n = tpw // ch
    mesh = plsc.VectorSubcoreMesh(core_axis_name="c", subcore_axis_name="s")

    @functools.partial(
        pl.kernel,
        out_type=jax.ShapeDtypeStruct((t, d), jnp.float32),
        mesh=mesh,
        scratch_types=[pltpu.VMEM((tpw,), jnp.int32),
                       pltpu.VMEM((tpw,), jnp.int32)]
        + [pltpu.VMEM((ch, d), jnp.float32)] * (2 * nbuf)
        + [pltpu.SemaphoreType.DMA] * (3 * nbuf),
    )
    def combine_k(y_hbm, p0_hbm, p1_hbm, out_hbm, i0_all, i1_all,
                  *bufs_sems):
        av = bufs_sems[:nbuf]
        bv = bufs_sems[nbuf:2 * nbuf]
        sems = bufs_sems[2 * nbuf:]
        sa = sems[:nbuf]
        sb = sems[nbuf:2 * nbuf]
        ss = sems[2 * nbuf:]
        wid = lax.axis_index("s") * NUM_CORES + lax.axis_index("c")
        base = wid * tpw
        pltpu.sync_copy(p0_hbm.at[pl.ds(base, tpw)], i0_all)
        pltpu.sync_copy(p1_hbm.at[pl.ds(base, tpw)], i1_all)
        ga = [None] * nbuf
        gb = [None] * nbuf
        st = [None] * nbuf

        def start(i):
            k = i % nbuf
            sl = pl.ds(i * ch, ch)
            ga[k] = pltpu.async_copy(y_hbm.at[i0_all.at[sl]], av[k], sa[k])
            gb[k] = pltpu.async_copy(y_hbm.at[i1_all.at[sl]], bv[k], sb[k])

        for i in range(min(nbuf, n)):
            start(i)
        for i in range(n):
            k = i % nbuf
            ga[k].wait()
            gb[k].wait()
            a_ref, b_ref = av[k], bv[k]

            def add_col(c, carry, a_ref=a_ref, b_ref=b_ref):
                sl = pl.ds(c * 16, 16)
                for r in range(ch):
                    a_ref[r, sl] = a_ref[r, sl] + b_ref[r, sl]
                return carry

            lax.fori_loop(0, d // 16, add_col, 0)
            st[k] = pltpu.async_copy(
                a_ref, out_hbm.at[pl.ds(base + i * ch, ch)], ss[k])
            j = i + nbuf
            if j < n:
                st[k].wait()
                start(j)
        for i in range(max(0, n - nbuf), n):
            st[i % nbuf].wait()

    return combine_k(ys, pos0, pos1)


def _mlp1_body(be_ref, xs_ref, w1_ref, w2_ref, h_ref):
    xb = xs_ref[...].astype(jnp.bfloat16)
    w1 = w1_ref[0].astype(jnp.bfloat16)
    w2 = w2_ref[0].astype(jnp.bfloat16)
    dn = (((1,), (1,)), ((), ()))
    h1 = lax.dot_general(xb, w1, dn, preferred_element_type=jnp.float32)
    h2 = lax.dot_general(xb, w2, dn, preferred_element_type=jnp.float32)
    h_ref[...] = (h1 * jax.nn.sigmoid(h1) * h2).astype(jnp.bfloat16)


def _mlp2_body(be_ref, h_ref, w3_ref, ps_ref, out_ref):
    hb = h_ref[...]
    w3 = w3_ref[0].astype(jnp.bfloat16)
    y = lax.dot_general(hb, w3, (((1,), (1,)), ((), ())),
                        preferred_element_type=jnp.float32)
    out_ref[...] = y * ps_ref[...]


def _grouped_mlp(block_expert, xs, fc1_w, fc2_w, fc3_w, row_scale):
    p, d = xs.shape
    e, f, _ = fc1_w.shape
    nblk = p // BLK
    hs = pl.pallas_call(
        _mlp1_body,
        grid_spec=pltpu.PrefetchScalarGridSpec(
            num_scalar_prefetch=1,
            grid=(nblk,),
            in_specs=[
                pl.BlockSpec((BLK, d), lambda b, be: (b, 0)),
                pl.BlockSpec((1, f, d), lambda b, be: (be[b], 0, 0)),
                pl.BlockSpec((1, f, d), lambda b, be: (be[b], 0, 0)),
            ],
            out_specs=pl.BlockSpec((BLK, f), lambda b, be: (b, 0)),
        ),
        out_shape=jax.ShapeDtypeStruct((p, f), jnp.bfloat16),
    )(block_expert, xs, fc1_w, fc2_w)
    return pl.pallas_call(
        _mlp2_body,
        grid_spec=pltpu.PrefetchScalarGridSpec(
            num_scalar_prefetch=1,
            grid=(nblk,),
            in_specs=[
                pl.BlockSpec((BLK, f), lambda b, be: (b, 0)),
                pl.BlockSpec((1, d, f), lambda b, be: (be[b], 0, 0)),
                pl.BlockSpec((BLK, 1), lambda b, be: (b, 0)),
            ],
            out_specs=pl.BlockSpec((BLK, d), lambda b, be: (b, 0)),
        ),
        out_shape=jax.ShapeDtypeStruct((p, d), jnp.float32),
    )(block_expert, hs, fc3_w, row_scale)


def kernel(x, gate_w, fc1_w, fc2_w, fc3_w):
    b, s, d = x.shape
    e = gate_w.shape[0]
    t = b * s
    p = TOP_K * t + e * BLK
    x2d = x.reshape(t, d)

    i0, i1, p0, p1 = _router(x2d, gate_w)

    e_flat = jnp.stack([i0, i1], axis=1).reshape(-1)
    p_flat = jnp.stack([p0, p1], axis=1).reshape(-1)
    oh = (e_flat[:, None] == jnp.arange(e, dtype=jnp.int32)[None, :]).astype(
        jnp.int32)
    cum = jnp.cumsum(oh, axis=0)
    rank = jnp.sum((cum - oh) * oh, axis=1)
    counts = cum[-1]
    padded = ((counts + BLK - 1) // BLK) * BLK
    starts = jnp.concatenate(
        [jnp.zeros((1,), jnp.int32), jnp.cumsum(padded)[:-1].astype(jnp.int32)])
    pos = starts[e_flat] + rank
    row_token = jnp.zeros((p,), jnp.int32).at[pos].set(
        jnp.repeat(jnp.arange(t, dtype=jnp.int32), TOP_K))
    row_scale = jnp.zeros((p, 1), jnp.float32).at[pos].set(p_flat[:, None])
    pos2 = pos.reshape(t, TOP_K).astype(jnp.int32)
    block_expert = (jnp.searchsorted(
        starts, jnp.arange(p // BLK, dtype=jnp.int32) * BLK, side="right")
        .astype(jnp.int32) - 1)

    xs = _sc_gather(x2d, row_token, p)
    ys = _grouped_mlp(block_expert, xs, fc1_w, fc2_w, fc3_w, row_scale)
    out2d = _sc_combine(ys, pos2[:, 0], pos2[:, 1])
    return out2d.reshape(b, s, d)
